# baseline (speedup 1.0000x reference)
import functools

import jax
import jax.numpy as jnp
from jax import lax
from jax.experimental import pallas as pl
from jax.experimental.pallas import tpu as pltpu
from jax.experimental.pallas import tpu_sc as plsc

N = 32
C = 3
H = W = 512
BLK = 16
G = H // BLK
NC, NS = 2, 16
GS = 16
GR = G - GS


def _sc_body(x_hbm, out_hbm, buf, out_v, sem_a, sem_b):
    cid = lax.axis_index("c")
    sid = lax.axis_index("s")
    n = sid * NC + cid

    def chunk_src(c, g0):
        return x_hbm.at[n, c, pl.ds(g0 * BLK, BLK), :]

    def issue(g0, base, sem):
        for c in range(C):
            pltpu.make_async_copy(chunk_src(c, g0), buf.at[base + c], sem).start()

    def process(g0, base, sem):
        for c in range(C):
            pltpu.make_async_copy(chunk_src(c, g0), buf.at[base + c], sem).wait()

        lane = lax.broadcasted_iota(jnp.int32, (16,), 0)
        for c in range(C):
            def rbody(r, acc, c=c):
                acc = list(acc)
                for g1 in range(G):
                    acc[g1] = acc[g1] + buf[base + c, r, pl.ds(g1 * BLK, BLK)]
                return tuple(acc)

            acc = lax.fori_loop(
                0, BLK, rbody,
                tuple(jnp.zeros((16,), jnp.float32) for _ in range(G)),
            )

            for h in range(2):
                vec = jnp.zeros((16,), jnp.float32)
                for j in range(16):
                    s = jnp.sum(acc[h * 16 + j])
                    vec = jnp.where(lane == j, s, vec)
                out_v[c, g0 - GS, pl.ds(h * 16, 16)] = vec

    issue(GS, 0, sem_a)
    issue(GS + 1, C, sem_b)
    npair = GR // 2

    def pbody(p, carry):
        g0a = GS + 2 * p
        process(g0a, 0, sem_a)

        @pl.when(p < npair - 1)
        def _():
            issue(g0a + 2, 0, sem_a)

        process(g0a + 1, C, sem_b)

        @pl.when(p < npair - 1)
        def _():
            issue(g0a + 3, C, sem_b)

        return carry

    lax.fori_loop(0, npair, pbody, 0)

    pltpu.sync_copy(out_v, out_hbm.at[n])


@jax.jit
def _sc_pool(x):
    mesh = plsc.VectorSubcoreMesh(core_axis_name="c", subcore_axis_name="s")
    return pl.kernel(
        _sc_body,
        out_type=jax.ShapeDtypeStruct((N, C, GR, G), jnp.float32),
        mesh=mesh,
        scratch_types=[
            pltpu.VMEM((2 * C, BLK, W), jnp.float32),
            pltpu.VMEM((C, GR, G), jnp.float32),
            pltpu.SemaphoreType.DMA,
            pltpu.SemaphoreType.DMA,
        ],
        compiler_params=pltpu.CompilerParams(needs_layout_passes=False),
    )(x)


GH = GS // 2


def _tc_pool_body(x_ref, b_ref, out_ref):
    ys = [
        jnp.sum(x_ref[0, c].reshape(GH, BLK, W), axis=1)
        for c in range(C)
    ]
    y = jnp.concatenate(ys, axis=0)
    z = jax.lax.dot(y, b_ref[...], precision=jax.lax.Precision.HIGHEST,
                    preferred_element_type=jnp.float32)
    out_ref[0] = z.reshape(C, GH, G)


@jax.jit
def _tc_pool(x, b):
    return pl.pallas_call(
        _tc_pool_body,
        grid=(N, 2),
        in_specs=[
            pl.BlockSpec((1, C, GH * BLK, W), lambda n, h: (n, 0, h, 0)),
            pl.BlockSpec((W, G), lambda n, h: (0, 0)),
        ],
        out_specs=pl.BlockSpec((1, C, GH, G), lambda n, h: (n, 0, h, 0)),
        out_shape=jax.ShapeDtypeStruct((N, C, GS, G), jnp.float32),
    )(x, b)


def _round_bf16(x):
    b = lax.bitcast_convert_type(x, jnp.uint32)
    b = (b + jnp.uint32(0x7FFF) + ((b >> 16) & jnp.uint32(1))) & jnp.uint32(0xFFFF0000)
    return lax.bitcast_convert_type(b, jnp.float32)


def _tc_body(st_ref, ss_ref, nz_ref, wb_ref, bp_ref,
             grid_ref, probs_ref, lp_ref, cnt_ref):
    st = st_ref[...]
    ss = ss_ref[...]
    m = [
        _round_bf16(
            jnp.concatenate([st[:, c], ss[:, c]], axis=1) * (1.0 / (BLK * BLK))
        )
        for c in range(C)
    ]
    logits = (m[0] * wb_ref[0] + m[1] * wb_ref[1]) + m[2] * wb_ref[2] + bp_ref[0]
    probs = jax.nn.sigmoid(logits)
    nz = nz_ref[...][:, 0]
    grid_f = jnp.where(nz < probs, 1.0, 0.0).astype(jnp.float32)
    lp = grid_f * jax.nn.log_sigmoid(logits) + (1.0 - grid_f) * jax.nn.log_sigmoid(-logits)
    grid_ref[...] = grid_f[:, None]
    probs_ref[...] = probs[:, None]
    lp_ref[...] = lp[:, None]
    cnt_ref[0] = jnp.sum(grid_f) * (1.0 / (N * G * G))


@jax.jit
def _tc_finish(st, ss, nz, wb, bp):
    shp = (N, 1, G, G)
    return pl.pallas_call(
        _tc_body,
        out_shape=(
            jax.ShapeDtypeStruct(shp, jnp.float32),
            jax.ShapeDtypeStruct(shp, jnp.float32),
            jax.ShapeDtypeStruct(shp, jnp.float32),
            jax.ShapeDtypeStruct((1,), jnp.float32),
        ),
        in_specs=[
            pl.BlockSpec(memory_space=pltpu.VMEM),
            pl.BlockSpec(memory_space=pltpu.VMEM),
            pl.BlockSpec(memory_space=pltpu.VMEM),
            pl.BlockSpec(memory_space=pltpu.SMEM),
            pl.BlockSpec(memory_space=pltpu.SMEM),
        ],
        out_specs=(
            pl.BlockSpec(memory_space=pltpu.VMEM),
            pl.BlockSpec(memory_space=pltpu.VMEM),
            pl.BlockSpec(memory_space=pltpu.VMEM),
            pl.BlockSpec(memory_space=pltpu.SMEM),
        ),
    )(st, ss, nz, wb, bp)


def _pool_mats():
    b = (jnp.arange(W)[:, None] // BLK == jnp.arange(G)[None, :]).astype(jnp.float32)
    return b


def kernel(inputs, noise, Wp, bp):
    b = _pool_mats()
    s_tc = _tc_pool(inputs, b)
    s_sc = _sc_pool(inputs)
    wb = lax.reduce_precision(Wp[0], 8, 7)

    grid_f, probs, lp, perc = _tc_finish(s_tc, s_sc, noise, wb, bp.reshape(1))
    return (
        grid_f.astype(bool),
        probs,
        lp,
        perc.reshape(()),
    )


# speedup vs baseline: 1.2785x; 1.2785x over previous
import functools

import jax
import jax.numpy as jnp
from jax import lax
from jax.experimental import pallas as pl
from jax.experimental.pallas import tpu as pltpu
from jax.experimental.pallas import tpu_sc as plsc

N = 32
C = 3
H = W = 512
BLK = 16
G = H // BLK
NC, NS = 2, 16
GS = 16
GR = G - GS


def _sc_body(x_hbm, out_hbm, buf, out_v, sem_a, sem_b):
    cid = lax.axis_index("c")
    sid = lax.axis_index("s")
    n = sid * NC + cid

    def chunk_src(c, g0):
        return x_hbm.at[n, c, pl.ds(g0 * BLK, BLK), :]

    def issue(g0, base, sem):
        for c in range(C):
            pltpu.make_async_copy(chunk_src(c, g0), buf.at[base + c], sem).start()

    def process(g0, base, sem):
        for c in range(C):
            pltpu.make_async_copy(chunk_src(c, g0), buf.at[base + c], sem).wait()

        lane = lax.broadcasted_iota(jnp.int32, (16,), 0)
        for c in range(C):
            def rbody(r, acc, c=c):
                acc = list(acc)
                for g1 in range(G):
                    acc[g1] = acc[g1] + buf[base + c, r, pl.ds(g1 * BLK, BLK)]
                return tuple(acc)

            acc = lax.fori_loop(
                0, BLK, rbody,
                tuple(jnp.zeros((16,), jnp.float32) for _ in range(G)),
            )

            for h in range(2):
                vec = jnp.zeros((16,), jnp.float32)
                for j in range(16):
                    s = jnp.sum(acc[h * 16 + j])
                    vec = jnp.where(lane == j, s, vec)
                out_v[c, g0 - GS, pl.ds(h * 16, 16)] = vec

    issue(GS, 0, sem_a)
    issue(GS + 1, C, sem_b)
    npair = GR // 2

    def pbody(p, carry):
        g0a = GS + 2 * p
        process(g0a, 0, sem_a)

        @pl.when(p < npair - 1)
        def _():
            issue(g0a + 2, 0, sem_a)

        process(g0a + 1, C, sem_b)

        @pl.when(p < npair - 1)
        def _():
            issue(g0a + 3, C, sem_b)

        return carry

    lax.fori_loop(0, npair, pbody, 0)

    pltpu.sync_copy(out_v, out_hbm.at[n])


@jax.jit
def _sc_pool(x):
    mesh = plsc.VectorSubcoreMesh(core_axis_name="c", subcore_axis_name="s")
    return pl.kernel(
        _sc_body,
        out_type=jax.ShapeDtypeStruct((N, C, GR, G), jnp.float32),
        mesh=mesh,
        scratch_types=[
            pltpu.VMEM((2 * C, BLK, W), jnp.float32),
            pltpu.VMEM((C, GR, G), jnp.float32),
            pltpu.SemaphoreType.DMA,
            pltpu.SemaphoreType.DMA,
        ],
        compiler_params=pltpu.CompilerParams(needs_layout_passes=False),
    )(x)


def _tc_pool_body(x_ref, b_ref, out_ref):
    ys = [
        jnp.sum(x_ref[0, c].reshape(GS, BLK, W), axis=1)
        for c in range(C)
    ]
    y = jnp.concatenate(ys, axis=0)
    z = jax.lax.dot(y, b_ref[...], precision=jax.lax.Precision.HIGHEST,
                    preferred_element_type=jnp.float32)
    out_ref[0] = z.reshape(C, GS, G)


@jax.jit
def _tc_pool(x, b):
    return pl.pallas_call(
        _tc_pool_body,
        grid=(N,),
        in_specs=[
            pl.BlockSpec((1, C, GS * BLK, W), lambda n: (n, 0, 0, 0)),
            pl.BlockSpec((W, G), lambda n: (0, 0)),
        ],
        out_specs=pl.BlockSpec((1, C, GS, G), lambda n: (n, 0, 0, 0)),
        out_shape=jax.ShapeDtypeStruct((N, C, GS, G), jnp.float32),
    )(x, b)


def _round_bf16(x):
    b = lax.bitcast_convert_type(x, jnp.uint32)
    b = (b + jnp.uint32(0x7FFF) + ((b >> 16) & jnp.uint32(1))) & jnp.uint32(0xFFFF0000)
    return lax.bitcast_convert_type(b, jnp.float32)


def _tc_body(st_ref, ss_ref, nz_ref, wb_ref, bp_ref,
             grid_ref, probs_ref, lp_ref, cnt_ref):
    st = st_ref[...]
    ss = ss_ref[...]
    m = [
        _round_bf16(
            jnp.concatenate([st[:, c], ss[:, c]], axis=1) * (1.0 / (BLK * BLK))
        )
        for c in range(C)
    ]
    logits = (m[0] * wb_ref[0] + m[1] * wb_ref[1]) + m[2] * wb_ref[2] + bp_ref[0]
    probs = jax.nn.sigmoid(logits)
    nz = nz_ref[...][:, 0]
    grid_f = jnp.where(nz < probs, 1.0, 0.0).astype(jnp.float32)
    lp = grid_f * jax.nn.log_sigmoid(logits) + (1.0 - grid_f) * jax.nn.log_sigmoid(-logits)
    grid_ref[...] = grid_f[:, None]
    probs_ref[...] = probs[:, None]
    lp_ref[...] = lp[:, None]
    cnt_ref[0] = jnp.sum(grid_f) * (1.0 / (N * G * G))


@jax.jit
def _tc_finish(st, ss, nz, wb, bp):
    shp = (N, 1, G, G)
    return pl.pallas_call(
        _tc_body,
        out_shape=(
            jax.ShapeDtypeStruct(shp, jnp.float32),
            jax.ShapeDtypeStruct(shp, jnp.float32),
            jax.ShapeDtypeStruct(shp, jnp.float32),
            jax.ShapeDtypeStruct((1,), jnp.float32),
        ),
        in_specs=[
            pl.BlockSpec(memory_space=pltpu.VMEM),
            pl.BlockSpec(memory_space=pltpu.VMEM),
            pl.BlockSpec(memory_space=pltpu.VMEM),
            pl.BlockSpec(memory_space=pltpu.SMEM),
            pl.BlockSpec(memory_space=pltpu.SMEM),
        ],
        out_specs=(
            pl.BlockSpec(memory_space=pltpu.VMEM),
            pl.BlockSpec(memory_space=pltpu.VMEM),
            pl.BlockSpec(memory_space=pltpu.VMEM),
            pl.BlockSpec(memory_space=pltpu.SMEM),
        ),
    )(st, ss, nz, wb, bp)


def _pool_mats():
    b = (jnp.arange(W)[:, None] // BLK == jnp.arange(G)[None, :]).astype(jnp.float32)
    return b


def kernel(inputs, noise, Wp, bp):
    b = _pool_mats()
    s_tc = _tc_pool(inputs, b)
    s_sc = _sc_pool(inputs)
    wb = lax.reduce_precision(Wp[0], 8, 7)

    grid_f, probs, lp, perc = _tc_finish(s_tc, s_sc, noise, wb, bp.reshape(1))
    return (
        grid_f.astype(bool),
        probs,
        lp,
        perc.reshape(()),
    )
